# baseline (speedup 1.0000x reference)
import functools

import jax
import jax.numpy as jnp
from jax import lax
from jax.experimental import pallas as pl
from jax.experimental.pallas import tpu as pltpu
from jax.experimental.pallas import tpu_sc as plsc

N = 10000
E = 320000
NP = 10240
NC = 2
NS = 16
NW = NC * NS
EPW = E // NW
CH = 80
NROW = E // CH
RPW = NROW // NW
NBUF = 4
ROWS_PER_TILE = NP // NS
DEGW = 16


def _zero_rows(ref, nrows, ncols):
    def row(i, _):
        def col(j, _):
            ref[i, pl.ds(j * 16, 16)] = jnp.zeros((16,), jnp.float32)
            return 0
        return lax.fori_loop(0, ncols // 16, col, 0)
    lax.fori_loop(0, nrows, row, 0)


def _fill_ones(ref, nrows, ncols):
    def row(i, _):
        def col(j, _):
            ref[i, pl.ds(j * 16, 16)] = jnp.ones((16,), jnp.float32)
            return 0
        return lax.fori_loop(0, ncols // 16, col, 0)
    lax.fori_loop(0, nrows, row, 0)


def _sc_mesh():
    return plsc.VectorSubcoreMesh(core_axis_name="c", subcore_axis_name="s")


def _deg_body(dst_hbm, out_hbm, didx, hist_v):
    c = lax.axis_index("c")
    s = lax.axis_index("s")
    wid = s * NC + c

    def zb(k, _):
        hist_v[k, pl.ds(0, 16)] = jnp.zeros((16,), jnp.float32)
        return 0
    lax.fori_loop(0, N // 16, zb, 0)

    ones16 = jnp.ones((16,), jnp.float32)

    def scat(iv):
        rows = lax.shift_right_logical(iv, 4)
        cols = lax.bitwise_and(iv, 15)
        plsc.addupdate_scatter(hist_v, [rows, cols], ones16)

    pltpu.sync_copy(dst_hbm.at[wid], didx)

    def body(j, _):
        def inner(k, _):
            scat(didx[j, pl.ds(k * 16, 16)])
            return 0
        return lax.fori_loop(0, CH // 16, inner, 0)
    lax.fori_loop(0, RPW, body, 0)

    pltpu.sync_copy(hist_v, out_hbm.at[wid])


def _sc_degree(dst3):
    return pl.kernel(
        _deg_body,
        out_type=jax.ShapeDtypeStruct((NW, N // 16, 16), jnp.float32),
        mesh=_sc_mesh(),
        scratch_types=[
            pltpu.VMEM((RPW, CH), jnp.int32),
            pltpu.VMEM((N // 16, 16), jnp.float32),
        ],
        compiler_params=pltpu.CompilerParams(needs_layout_passes=False),
    )(dst3)


def _agg_body(g_hbm, src_hbm, dst_hbm, out_hbm, *scratch, d):
    sidxs = scratch[:NBUF]
    didxs = scratch[NBUF:2 * NBUF]
    rowss = scratch[2 * NBUF:3 * NBUF]
    acc_sh = scratch[3 * NBUF]
    semi = scratch[3 * NBUF + 1:3 * NBUF + 1 + NBUF]
    semr = scratch[3 * NBUF + 1 + NBUF:]
    rows0 = rowss[0]
    c = lax.axis_index("c")
    s = lax.axis_index("s")
    wid = s * NC + c

    _zero_rows(rows0, CH, d)
    for k in range(ROWS_PER_TILE // CH):
        pltpu.sync_copy(rows0, acc_sh.at[pl.ds(s * ROWS_PER_TILE + k * CH, CH)])
    plsc.subcore_barrier()

    def start_idx(p, t):
        pltpu.async_copy(src_hbm.at[wid, pl.ds(p, 1)], sidxs[t], semi[t])
        pltpu.async_copy(dst_hbm.at[wid, pl.ds(p, 1)], didxs[t], semi[t])

    def start_gather(g, t):
        pltpu.make_async_copy(src_hbm.at[wid, pl.ds(g, 1)], sidxs[t],
                              semi[t]).wait()
        pltpu.make_async_copy(dst_hbm.at[wid, pl.ds(g, 1)], didxs[t],
                              semi[t]).wait()
        pltpu.async_copy(g_hbm.at[sidxs[t].at[0]], rowss[t], semr[t])

    def drain(j, t):
        pltpu.make_async_copy(g_hbm.at[sidxs[t].at[0]], rowss[t],
                              semr[t]).wait()
        pltpu.sync_copy(rowss[t], acc_sh.at[didxs[t].at[0]], add=True)

    for t in range(NBUF):
        start_idx(t, t)
    for t in range(NBUF - 1):
        start_gather(t, t)

    def body(jb, _):
        j = jb * NBUF
        for t in range(NBUF):
            jj = j + t

            @pl.when(jj < RPW)
            def _(jj=jj, t=t):
                @pl.when(jj + NBUF - 1 < RPW)
                def _():
                    start_gather(jj + NBUF - 1, (t + NBUF - 1) % NBUF)
                drain(jj, t)

                @pl.when(jj + NBUF < RPW)
                def _():
                    start_idx(jj + NBUF, t)
        return 0
    lax.fori_loop(0, (RPW + NBUF - 1) // NBUF, body, 0)

    plsc.subcore_barrier()

    for k in range(ROWS_PER_TILE // CH):
        base = s * ROWS_PER_TILE + k * CH
        pltpu.sync_copy(acc_sh.at[pl.ds(base, CH)], rows0)
        pltpu.sync_copy(rows0, out_hbm.at[c, pl.ds(base, CH)])


def _sc_agg(g, src3, dst3, d):
    cp = None if d == 128 else pltpu.CompilerParams(use_tc_tiling_on_sc=False)
    return pl.kernel(
        functools.partial(_agg_body, d=d),
        compiler_params=cp,
        out_type=jax.ShapeDtypeStruct((NC, NP, d), jnp.float32),
        mesh=_sc_mesh(),
        scratch_types=(
            [pltpu.VMEM((1, CH), jnp.int32) for _ in range(2 * NBUF)]
            + [pltpu.VMEM((CH, d), jnp.float32) for _ in range(NBUF)]
            + [pltpu.VMEM_SHARED((NP, d), jnp.float32)]
            + [pltpu.SemaphoreType.DMA for _ in range(2 * NBUF)]
        ),
    )(g, src3, dst3)


R = 1000


def _tc1_body(x_ref, w1_ref, h1_ref):
    h1_ref[...] = jnp.dot(x_ref[...], w1_ref[...],
                          preferred_element_type=jnp.float32)


def _tc1(x, W1):
    grid = N // R
    return pl.pallas_call(
        _tc1_body,
        grid=(grid,),
        in_specs=[
            pl.BlockSpec((R, 128), lambda i: (i, 0)),
            pl.BlockSpec((128, 128), lambda i: (0, 0)),
        ],
        out_specs=pl.BlockSpec((R, 128), lambda i: (i, 0)),
        out_shape=jax.ShapeDtypeStruct((N, 128), jnp.float32),
    )(x, W1)


def _dinv_scale_body(dp_ref, h1_ref, dinv_ref, g1_ref):
    deg = jnp.sum(dp_ref[...], axis=0) + 1.0
    dinv = lax.rsqrt(deg)[:, None]
    dinv_ref[...] = dinv
    g1_ref[...] = h1_ref[...] * dinv


def _dinv_scale(dp, h1):
    return pl.pallas_call(
        _dinv_scale_body,
        out_shape=[
            jax.ShapeDtypeStruct((N, 1), jnp.float32),
            jax.ShapeDtypeStruct((N, 128), jnp.float32),
        ],
    )(dp, h1)


def _tc2_body(dinv_ref, agg_ref, g1_ref, b1_ref, w2_ref, g2_ref):
    dinv = dinv_ref[...]
    out1 = dinv * (agg_ref[0] + agg_ref[1] + g1_ref[...]) + b1_ref[...]
    out1 = jnp.maximum(out1, 0.0)
    g2_ref[...] = jnp.dot(out1, w2_ref[...],
                          preferred_element_type=jnp.float32) * dinv


def _tc2(dinv, agg1, g1, b1, W2):
    grid = N // R
    return pl.pallas_call(
        _tc2_body,
        grid=(grid,),
        in_specs=[
            pl.BlockSpec((R, 1), lambda i: (i, 0)),
            pl.BlockSpec((NC, R, 128), lambda i: (0, i, 0)),
            pl.BlockSpec((R, 128), lambda i: (i, 0)),
            pl.BlockSpec((1, 128), lambda i: (0, 0)),
            pl.BlockSpec((128, 64), lambda i: (0, 0)),
        ],
        out_specs=pl.BlockSpec((R, 64), lambda i: (i, 0)),
        out_shape=jax.ShapeDtypeStruct((N, 64), jnp.float32),
    )(dinv, agg1, g1, b1.reshape(1, 128), W2)


def _tc3_body(dinv_ref, agg_ref, g2_ref, b2_ref, wd_ref, bd_ref, pred_ref):
    dinv = dinv_ref[...]
    z = dinv * (agg_ref[0] + agg_ref[1] + g2_ref[...]) + b2_ref[...]
    m = jnp.max(z, axis=1, keepdims=True)
    lse = jnp.log(jnp.sum(jnp.exp(z - m), axis=1, keepdims=True)) + m
    embeds = z - lse
    logit = jnp.sum(embeds * wd_ref[...], axis=1, keepdims=True) + bd_ref[0, 0]
    pred_ref[...] = jax.nn.sigmoid(logit)


def _tc3(dinv, agg2, g2, b2, Wd, bd):
    grid = N // R
    return pl.pallas_call(
        _tc3_body,
        grid=(grid,),
        in_specs=[
            pl.BlockSpec((R, 1), lambda i: (i, 0)),
            pl.BlockSpec((NC, R, 64), lambda i: (0, i, 0)),
            pl.BlockSpec((R, 64), lambda i: (i, 0)),
            pl.BlockSpec((1, 64), lambda i: (0, 0)),
            pl.BlockSpec((1, 64), lambda i: (0, 0)),
            pl.BlockSpec((1, 1), lambda i: (0, 0)),
        ],
        out_specs=pl.BlockSpec((R, 1), lambda i: (i, 0)),
        out_shape=jax.ShapeDtypeStruct((N, 1), jnp.float32),
    )(dinv, agg2, g2, b2.reshape(1, 64), Wd.reshape(1, 64), bd.reshape(1, 1))


def kernel(x, edge_index, W1, b1, W2, b2, Wd, bd):
    src3 = edge_index[0].reshape(NW, RPW, CH)
    dst3 = edge_index[1].reshape(NW, RPW, CH)
    dp = _sc_degree(dst3).reshape(NW, N)
    h1 = _tc1(x, W1)
    dinv, g1 = _dinv_scale(dp, h1)
    agg1 = _sc_agg(g1, src3, dst3, 128)
    g2 = _tc2(dinv, agg1, g1, b1, W2)
    agg2 = _sc_agg(g2, src3, dst3, 64)
    return _tc3(dinv, agg2, g2, b2, Wd, bd)


# speedup vs baseline: 1.3137x; 1.3137x over previous
import functools

import jax
import jax.numpy as jnp
from jax import lax
from jax.experimental import pallas as pl
from jax.experimental.pallas import tpu as pltpu
from jax.experimental.pallas import tpu_sc as plsc

N = 10000
E = 320000
NP = 10240
NC = 2
NS = 16
NW = NC * NS
EPW = E // NW
CH = 80
NROW = E // CH
RPW = NROW // NW
NBUF = 3
ROWS_PER_TILE = NP // NS
DEGW = 16


def _zero_rows(ref, nrows, ncols):
    def row(i, _):
        def col(j, _):
            ref[i, pl.ds(j * 16, 16)] = jnp.zeros((16,), jnp.float32)
            return 0
        return lax.fori_loop(0, ncols // 16, col, 0)
    lax.fori_loop(0, nrows, row, 0)


def _fill_ones(ref, nrows, ncols):
    def row(i, _):
        def col(j, _):
            ref[i, pl.ds(j * 16, 16)] = jnp.ones((16,), jnp.float32)
            return 0
        return lax.fori_loop(0, ncols // 16, col, 0)
    lax.fori_loop(0, nrows, row, 0)


def _sc_mesh():
    return plsc.VectorSubcoreMesh(core_axis_name="c", subcore_axis_name="s")


def _deg_body(dst_hbm, out_hbm, didx, hist_v):
    c = lax.axis_index("c")
    s = lax.axis_index("s")
    wid = s * NC + c

    def zb(k, _):
        hist_v[k, pl.ds(0, 16)] = jnp.zeros((16,), jnp.float32)
        return 0
    lax.fori_loop(0, N // 16, zb, 0)

    ones16 = jnp.ones((16,), jnp.float32)

    def scat(iv):
        rows = lax.shift_right_logical(iv, 4)
        cols = lax.bitwise_and(iv, 15)
        plsc.addupdate_scatter(hist_v, [rows, cols], ones16)

    pltpu.sync_copy(dst_hbm.at[wid], didx)

    def body(j, _):
        def inner(k, _):
            scat(didx[j, pl.ds(k * 16, 16)])
            return 0
        return lax.fori_loop(0, CH // 16, inner, 0)
    lax.fori_loop(0, RPW, body, 0)

    pltpu.sync_copy(hist_v, out_hbm.at[wid])


def _sc_degree(dst3):
    return pl.kernel(
        _deg_body,
        out_type=jax.ShapeDtypeStruct((NW, N // 16, 16), jnp.float32),
        mesh=_sc_mesh(),
        scratch_types=[
            pltpu.VMEM((RPW, CH), jnp.int32),
            pltpu.VMEM((N // 16, 16), jnp.float32),
        ],
        compiler_params=pltpu.CompilerParams(needs_layout_passes=False),
    )(dst3)


def _agg_body(g_hbm, src_hbm, dst_hbm, out_hbm, sidx, *scratch, d):
    didxs = scratch[:NBUF]
    rowss = scratch[NBUF:2 * NBUF]
    acc_sh = scratch[2 * NBUF]
    sems = scratch[2 * NBUF + 1:]
    rows0 = rowss[0]
    c = lax.axis_index("c")
    s = lax.axis_index("s")
    wid = s * NC + c

    _zero_rows(rows0, CH, d)
    for k in range(ROWS_PER_TILE // CH):
        pltpu.sync_copy(rows0, acc_sh.at[pl.ds(s * ROWS_PER_TILE + k * CH, CH)])
    plsc.subcore_barrier()

    pltpu.sync_copy(src_hbm.at[wid], sidx)

    def start(j, rows, didx, sem):
        pltpu.async_copy(dst_hbm.at[wid, pl.ds(j, 1)], didx, sem)
        pltpu.async_copy(g_hbm.at[sidx.at[j]], rows, sem)

    def drain(j, rows, didx, sem):
        pltpu.make_async_copy(dst_hbm.at[wid, pl.ds(j, 1)], didx, sem).wait()
        pltpu.make_async_copy(g_hbm.at[sidx.at[j]], rows, sem).wait()
        pltpu.sync_copy(rows, acc_sh.at[didx.at[0]], add=True)

    bufs = tuple((rowss[t], didxs[t], sems[t]) for t in range(NBUF))
    for t in range(NBUF):
        start(t, *bufs[t])

    def body(j3, _):
        j = j3 * NBUF
        for t in range(NBUF):
            @pl.when(j + t < RPW)
            def _(t=t):
                drain(j + t, *bufs[t])

                @pl.when(j + t + NBUF < RPW)
                def _():
                    start(j + t + NBUF, *bufs[t])
        return 0
    lax.fori_loop(0, (RPW + NBUF - 1) // NBUF, body, 0)

    plsc.subcore_barrier()

    for k in range(ROWS_PER_TILE // CH):
        base = s * ROWS_PER_TILE + k * CH
        pltpu.sync_copy(acc_sh.at[pl.ds(base, CH)], rows0)
        pltpu.sync_copy(rows0, out_hbm.at[c, pl.ds(base, CH)])


def _sc_agg(g, src3, dst3, d):
    cp = None if d == 128 else pltpu.CompilerParams(use_tc_tiling_on_sc=False)
    return pl.kernel(
        functools.partial(_agg_body, d=d),
        compiler_params=cp,
        out_type=jax.ShapeDtypeStruct((NC, NP, d), jnp.float32),
        mesh=_sc_mesh(),
        scratch_types=(
            [pltpu.VMEM((RPW, CH), jnp.int32)]
            + [pltpu.VMEM((1, CH), jnp.int32) for _ in range(NBUF)]
            + [pltpu.VMEM((CH, d), jnp.float32) for _ in range(NBUF)]
            + [pltpu.VMEM_SHARED((NP, d), jnp.float32)]
            + [pltpu.SemaphoreType.DMA for _ in range(NBUF)]
        ),
    )(g, src3, dst3)


R = 1000


def _tc1_body(x_ref, w1_ref, h1_ref):
    h1_ref[...] = jnp.dot(x_ref[...], w1_ref[...],
                          preferred_element_type=jnp.float32)


def _tc1(x, W1):
    grid = N // R
    return pl.pallas_call(
        _tc1_body,
        grid=(grid,),
        in_specs=[
            pl.BlockSpec((R, 128), lambda i: (i, 0)),
            pl.BlockSpec((128, 128), lambda i: (0, 0)),
        ],
        out_specs=pl.BlockSpec((R, 128), lambda i: (i, 0)),
        out_shape=jax.ShapeDtypeStruct((N, 128), jnp.float32),
    )(x, W1)


def _dinv_scale_body(dp_ref, h1_ref, dinv_ref, g1_ref):
    deg = jnp.sum(dp_ref[...], axis=0) + 1.0
    dinv = lax.rsqrt(deg)[:, None]
    dinv_ref[...] = dinv
    g1_ref[...] = h1_ref[...] * dinv


def _dinv_scale(dp, h1):
    return pl.pallas_call(
        _dinv_scale_body,
        out_shape=[
            jax.ShapeDtypeStruct((N, 1), jnp.float32),
            jax.ShapeDtypeStruct((N, 128), jnp.float32),
        ],
    )(dp, h1)


def _tc2_body(dinv_ref, agg_ref, g1_ref, b1_ref, w2_ref, g2_ref):
    dinv = dinv_ref[...]
    out1 = dinv * (agg_ref[0] + agg_ref[1] + g1_ref[...]) + b1_ref[...]
    out1 = jnp.maximum(out1, 0.0)
    g2_ref[...] = jnp.dot(out1, w2_ref[...],
                          preferred_element_type=jnp.float32) * dinv


def _tc2(dinv, agg1, g1, b1, W2):
    grid = N // R
    return pl.pallas_call(
        _tc2_body,
        grid=(grid,),
        in_specs=[
            pl.BlockSpec((R, 1), lambda i: (i, 0)),
            pl.BlockSpec((NC, R, 128), lambda i: (0, i, 0)),
            pl.BlockSpec((R, 128), lambda i: (i, 0)),
            pl.BlockSpec((1, 128), lambda i: (0, 0)),
            pl.BlockSpec((128, 64), lambda i: (0, 0)),
        ],
        out_specs=pl.BlockSpec((R, 64), lambda i: (i, 0)),
        out_shape=jax.ShapeDtypeStruct((N, 64), jnp.float32),
    )(dinv, agg1, g1, b1.reshape(1, 128), W2)


def _tc3_body(dinv_ref, agg_ref, g2_ref, b2_ref, wd_ref, bd_ref, pred_ref):
    dinv = dinv_ref[...]
    z = dinv * (agg_ref[0] + agg_ref[1] + g2_ref[...]) + b2_ref[...]
    m = jnp.max(z, axis=1, keepdims=True)
    lse = jnp.log(jnp.sum(jnp.exp(z - m), axis=1, keepdims=True)) + m
    embeds = z - lse
    logit = jnp.sum(embeds * wd_ref[...], axis=1, keepdims=True) + bd_ref[0, 0]
    pred_ref[...] = jax.nn.sigmoid(logit)


def _tc3(dinv, agg2, g2, b2, Wd, bd):
    grid = N // R
    return pl.pallas_call(
        _tc3_body,
        grid=(grid,),
        in_specs=[
            pl.BlockSpec((R, 1), lambda i: (i, 0)),
            pl.BlockSpec((NC, R, 64), lambda i: (0, i, 0)),
            pl.BlockSpec((R, 64), lambda i: (i, 0)),
            pl.BlockSpec((1, 64), lambda i: (0, 0)),
            pl.BlockSpec((1, 64), lambda i: (0, 0)),
            pl.BlockSpec((1, 1), lambda i: (0, 0)),
        ],
        out_specs=pl.BlockSpec((R, 1), lambda i: (i, 0)),
        out_shape=jax.ShapeDtypeStruct((N, 1), jnp.float32),
    )(dinv, agg2, g2, b2.reshape(1, 64), Wd.reshape(1, 64), bd.reshape(1, 1))


def kernel(x, edge_index, W1, b1, W2, b2, Wd, bd):
    src3 = edge_index[0].reshape(NW, RPW, CH)
    dst3 = edge_index[1].reshape(NW, RPW, CH)
    dp = _sc_degree(dst3).reshape(NW, N)
    h1 = _tc1(x, W1)
    dinv, g1 = _dinv_scale(dp, h1)
    agg1 = _sc_agg(g1, src3, dst3, 128)
    g2 = _tc2(dinv, agg1, g1, b1, W2)
    agg2 = _sc_agg(g2, src3, dst3, 64)
    return _tc3(dinv, agg2, g2, b2, Wd, bd)


# speedup vs baseline: 1.3468x; 1.0252x over previous
import functools

import jax
import jax.numpy as jnp
from jax import lax
from jax.experimental import pallas as pl
from jax.experimental.pallas import tpu as pltpu
from jax.experimental.pallas import tpu_sc as plsc

N = 10000
E = 320000
NP = 10240
NC = 2
NS = 16
NW = NC * NS
EPW = E // NW
CH = 80
NROW = E // CH
RPW = NROW // NW
NBUF = 3
ROWS_PER_TILE = NP // NS
DEGW = 16


def _zero_rows(ref, nrows, ncols):
    def row(i, _):
        def col(j, _):
            ref[i, pl.ds(j * 16, 16)] = jnp.zeros((16,), jnp.float32)
            return 0
        return lax.fori_loop(0, ncols // 16, col, 0)
    lax.fori_loop(0, nrows, row, 0)


def _fill_ones(ref, nrows, ncols):
    def row(i, _):
        def col(j, _):
            ref[i, pl.ds(j * 16, 16)] = jnp.ones((16,), jnp.float32)
            return 0
        return lax.fori_loop(0, ncols // 16, col, 0)
    lax.fori_loop(0, nrows, row, 0)


def _sc_mesh():
    return plsc.VectorSubcoreMesh(core_axis_name="c", subcore_axis_name="s")


def _deg_body(dst_hbm, out_hbm, didx, hist_v):
    c = lax.axis_index("c")
    s = lax.axis_index("s")
    wid = s * NC + c

    def zb(k, _):
        hist_v[k, pl.ds(0, 16)] = jnp.zeros((16,), jnp.float32)
        return 0
    lax.fori_loop(0, N // 16, zb, 0)

    ones16 = jnp.ones((16,), jnp.float32)

    def scat(iv):
        rows = lax.shift_right_logical(iv, 4)
        cols = lax.bitwise_and(iv, 15)
        plsc.addupdate_scatter(hist_v, [rows, cols], ones16)

    pltpu.sync_copy(dst_hbm.at[wid], didx)

    def body(j, _):
        def inner(k, _):
            scat(didx[j, pl.ds(k * 16, 16)])
            return 0
        return lax.fori_loop(0, CH // 16, inner, 0)
    lax.fori_loop(0, RPW, body, 0)

    pltpu.sync_copy(hist_v, out_hbm.at[wid])


def _sc_degree(dst3):
    return pl.kernel(
        _deg_body,
        out_type=jax.ShapeDtypeStruct((NW, N // 16, 16), jnp.float32),
        mesh=_sc_mesh(),
        scratch_types=[
            pltpu.VMEM((RPW, CH), jnp.int32),
            pltpu.VMEM((N // 16, 16), jnp.float32),
        ],
        compiler_params=pltpu.CompilerParams(needs_layout_passes=False),
    )(dst3)


def _agg_body(g_hbm, src_hbm, dst_hbm, out_hbm, sidx, *scratch, d):
    didxs = scratch[:NBUF]
    rowss = scratch[NBUF:2 * NBUF]
    acc_sh = scratch[2 * NBUF]
    sems = scratch[2 * NBUF + 1:]
    rows0 = rowss[0]
    c = lax.axis_index("c")
    s = lax.axis_index("s")
    wid = s * NC + c

    pltpu.sync_copy(src_hbm.at[wid], sidx)

    def start(j, rows, didx, sem):
        pltpu.async_copy(dst_hbm.at[wid, pl.ds(j, 1)], didx, sem)
        pltpu.async_copy(g_hbm.at[sidx.at[j]], rows, sem)

    def drain(j, rows, didx, sem):
        pltpu.make_async_copy(dst_hbm.at[wid, pl.ds(j, 1)], didx, sem).wait()
        pltpu.make_async_copy(g_hbm.at[sidx.at[j]], rows, sem).wait()
        pltpu.sync_copy(rows, acc_sh.at[didx.at[0]], add=True)

    bufs = tuple((rowss[t], didxs[t], sems[t]) for t in range(NBUF))

    _zero_rows(rowss[1], CH, d)
    start(0, *bufs[0])
    for k in range(ROWS_PER_TILE // CH):
        pltpu.async_copy(
            rowss[1], acc_sh.at[pl.ds(s * ROWS_PER_TILE + k * CH, CH)],
            sems[1])
    start(2, *bufs[2])
    for k in range(ROWS_PER_TILE // CH):
        pltpu.make_async_copy(
            rowss[1], acc_sh.at[pl.ds(s * ROWS_PER_TILE + k * CH, CH)],
            sems[1]).wait()
    start(1, *bufs[1])
    plsc.subcore_barrier()

    def body(j3, _):
        j = j3 * NBUF
        for t in range(NBUF):
            @pl.when(j + t < RPW)
            def _(t=t):
                drain(j + t, *bufs[t])

                @pl.when(j + t + NBUF < RPW)
                def _():
                    start(j + t + NBUF, *bufs[t])
        return 0
    lax.fori_loop(0, (RPW + NBUF - 1) // NBUF, body, 0)

    plsc.subcore_barrier()

    for k in range(ROWS_PER_TILE // CH):
        base = s * ROWS_PER_TILE + k * CH
        rb, sb = rowss[k % 2], sems[k % 2]
        if k >= 2:
            prev = s * ROWS_PER_TILE + (k - 2) * CH
            pltpu.make_async_copy(rb, out_hbm.at[c, pl.ds(prev, CH)],
                                  sb).wait()
        pltpu.sync_copy(acc_sh.at[pl.ds(base, CH)], rb)
        pltpu.async_copy(rb, out_hbm.at[c, pl.ds(base, CH)], sb)
    for k in range(ROWS_PER_TILE // CH - 2, ROWS_PER_TILE // CH):
        base = s * ROWS_PER_TILE + k * CH
        pltpu.make_async_copy(rowss[k % 2], out_hbm.at[c, pl.ds(base, CH)],
                              sems[k % 2]).wait()


def _sc_agg(g, src3, dst3, d):
    cp = None if d == 128 else pltpu.CompilerParams(use_tc_tiling_on_sc=False)
    return pl.kernel(
        functools.partial(_agg_body, d=d),
        compiler_params=cp,
        out_type=jax.ShapeDtypeStruct((NC, NP, d), jnp.float32),
        mesh=_sc_mesh(),
        scratch_types=(
            [pltpu.VMEM((RPW, CH), jnp.int32)]
            + [pltpu.VMEM((1, CH), jnp.int32) for _ in range(NBUF)]
            + [pltpu.VMEM((CH, d), jnp.float32) for _ in range(NBUF)]
            + [pltpu.VMEM_SHARED((NP, d), jnp.float32)]
            + [pltpu.SemaphoreType.DMA for _ in range(NBUF)]
        ),
    )(g, src3, dst3)


R = 1000


def _tc1_body(x_ref, w1_ref, h1_ref):
    h1_ref[...] = jnp.dot(x_ref[...], w1_ref[...],
                          preferred_element_type=jnp.float32)


def _tc1(x, W1):
    grid = N // R
    return pl.pallas_call(
        _tc1_body,
        grid=(grid,),
        in_specs=[
            pl.BlockSpec((R, 128), lambda i: (i, 0)),
            pl.BlockSpec((128, 128), lambda i: (0, 0)),
        ],
        out_specs=pl.BlockSpec((R, 128), lambda i: (i, 0)),
        out_shape=jax.ShapeDtypeStruct((N, 128), jnp.float32),
    )(x, W1)


def _dinv_scale_body(dp_ref, h1_ref, dinv_ref, g1_ref):
    deg = jnp.sum(dp_ref[...], axis=0) + 1.0
    dinv = lax.rsqrt(deg)[:, None]
    dinv_ref[...] = dinv
    g1_ref[...] = h1_ref[...] * dinv


def _dinv_scale(dp, h1):
    return pl.pallas_call(
        _dinv_scale_body,
        out_shape=[
            jax.ShapeDtypeStruct((N, 1), jnp.float32),
            jax.ShapeDtypeStruct((N, 128), jnp.float32),
        ],
    )(dp, h1)


def _tc2_body(dinv_ref, agg_ref, g1_ref, b1_ref, w2_ref, g2_ref):
    dinv = dinv_ref[...]
    out1 = dinv * (agg_ref[0] + agg_ref[1] + g1_ref[...]) + b1_ref[...]
    out1 = jnp.maximum(out1, 0.0)
    g2_ref[...] = jnp.dot(out1, w2_ref[...],
                          preferred_element_type=jnp.float32) * dinv


def _tc2(dinv, agg1, g1, b1, W2):
    grid = N // R
    return pl.pallas_call(
        _tc2_body,
        grid=(grid,),
        in_specs=[
            pl.BlockSpec((R, 1), lambda i: (i, 0)),
            pl.BlockSpec((NC, R, 128), lambda i: (0, i, 0)),
            pl.BlockSpec((R, 128), lambda i: (i, 0)),
            pl.BlockSpec((1, 128), lambda i: (0, 0)),
            pl.BlockSpec((128, 64), lambda i: (0, 0)),
        ],
        out_specs=pl.BlockSpec((R, 64), lambda i: (i, 0)),
        out_shape=jax.ShapeDtypeStruct((N, 64), jnp.float32),
    )(dinv, agg1, g1, b1.reshape(1, 128), W2)


def _tc3_body(dinv_ref, agg_ref, g2_ref, b2_ref, wd_ref, bd_ref, pred_ref):
    dinv = dinv_ref[...]
    z = dinv * (agg_ref[0] + agg_ref[1] + g2_ref[...]) + b2_ref[...]
    m = jnp.max(z, axis=1, keepdims=True)
    lse = jnp.log(jnp.sum(jnp.exp(z - m), axis=1, keepdims=True)) + m
    embeds = z - lse
    logit = jnp.sum(embeds * wd_ref[...], axis=1, keepdims=True) + bd_ref[0, 0]
    pred_ref[...] = jax.nn.sigmoid(logit)


def _tc3(dinv, agg2, g2, b2, Wd, bd):
    grid = N // R
    return pl.pallas_call(
        _tc3_body,
        grid=(grid,),
        in_specs=[
            pl.BlockSpec((R, 1), lambda i: (i, 0)),
            pl.BlockSpec((NC, R, 64), lambda i: (0, i, 0)),
            pl.BlockSpec((R, 64), lambda i: (i, 0)),
            pl.BlockSpec((1, 64), lambda i: (0, 0)),
            pl.BlockSpec((1, 64), lambda i: (0, 0)),
            pl.BlockSpec((1, 1), lambda i: (0, 0)),
        ],
        out_specs=pl.BlockSpec((R, 1), lambda i: (i, 0)),
        out_shape=jax.ShapeDtypeStruct((N, 1), jnp.float32),
    )(dinv, agg2, g2, b2.reshape(1, 64), Wd.reshape(1, 64), bd.reshape(1, 1))


def kernel(x, edge_index, W1, b1, W2, b2, Wd, bd):
    src3 = edge_index[0].reshape(NW, RPW, CH)
    dst3 = edge_index[1].reshape(NW, RPW, CH)
    dp = _sc_degree(dst3).reshape(NW, N)
    h1 = _tc1(x, W1)
    dinv, g1 = _dinv_scale(dp, h1)
    agg1 = _sc_agg(g1, src3, dst3, 128)
    g2 = _tc2(dinv, agg1, g1, b1, W2)
    agg2 = _sc_agg(g2, src3, dst3, 64)
    return _tc3(dinv, agg2, g2, b2, Wd, bd)
